# Initial kernel scaffold; baseline (speedup 1.0000x reference)
import jax
import jax.numpy as jnp
from jax.experimental import pallas as pl


def kernel(x, edge_index, W1, b1, W3, b3, W2, b2):
    raise NotImplementedError("write your pallas kernel here")


# speedup vs baseline: 32.8371x; 32.8371x over previous
import functools

import jax
import jax.numpy as jnp
from jax import lax
from jax.experimental import pallas as pl
from jax.experimental.pallas import tpu as pltpu
from jax.experimental.pallas import tpu_sc as plsc

N_CORES = 2
N_SUBCORES = 16
N_WORKERS = N_CORES * N_SUBCORES
CHUNK = 128
K_INNER = 8

N_NODES = 100000
N_PAD = 100096
E_EDGES = 3200000
E_PAD = 3211264
E_ROWS = E_PAD // CHUNK
R_PER_W = E_ROWS // N_WORKERS
N_BLK = R_PER_W // K_INNER


def _worker_mesh():
    return plsc.VectorSubcoreMesh(core_axis_name="c", subcore_axis_name="s")


def _make_prop(d):
    @functools.partial(
        pl.kernel,
        out_type=jax.ShapeDtypeStruct((N_CORES, N_PAD, d), jnp.float32),
        mesh=_worker_mesh(),
        compiler_params=pltpu.CompilerParams(use_tc_tiling_on_sc=False),
        scratch_types=[
            pltpu.VMEM((K_INNER, CHUNK), jnp.int32),
            pltpu.VMEM((K_INNER, CHUNK), jnp.int32),
            pltpu.VMEM((K_INNER * CHUNK, d), jnp.float32),
            pltpu.VMEM_SHARED((N_PAD, d), jnp.float32),
            pltpu.SemaphoreType.DMA,
            pltpu.SemaphoreType.DMA,
        ],
    )
    def prop(u_hbm, zeros_hbm, src_hbm, dst_hbm, out_hbm,
             srcb, dstb, rows, acc, gsem, isem):
        c = lax.axis_index("c")
        s = lax.axis_index("s")
        wid = c * N_SUBCORES + s
        zr = N_PAD // N_SUBCORES
        pltpu.sync_copy(zeros_hbm.at[pl.ds(s * zr, zr)],
                        acc.at[pl.ds(s * zr, zr)])
        plsc.subcore_barrier()
        row0 = wid * R_PER_W

        def body(b, carry):
            r0 = row0 + b * K_INNER
            cps = pltpu.async_copy(src_hbm.at[pl.ds(r0, K_INNER)], srcb, isem)
            cpd = pltpu.async_copy(dst_hbm.at[pl.ds(r0, K_INNER)], dstb, isem)
            cps.wait()
            cpd.wait()
            gathers = []
            for j in range(K_INNER):
                gathers.append(pltpu.async_copy(
                    u_hbm.at[srcb.at[j]],
                    rows.at[pl.ds(j * CHUNK, CHUNK)], gsem))
            for j in range(K_INNER):
                gathers[j].wait()
                pltpu.sync_copy(rows.at[pl.ds(j * CHUNK, CHUNK)],
                                acc.at[dstb.at[j]], add=True)
            return carry

        lax.fori_loop(0, N_BLK, body, 0)
        plsc.subcore_barrier()
        pltpu.sync_copy(acc.at[pl.ds(s * zr, zr)],
                        out_hbm.at[c, pl.ds(s * zr, zr)])

    return prop


def _make_deg():
    @functools.partial(
        pl.kernel,
        out_type=jax.ShapeDtypeStruct((N_CORES, N_PAD, 1), jnp.float32),
        mesh=_worker_mesh(),
        compiler_params=pltpu.CompilerParams(use_tc_tiling_on_sc=False),
        scratch_types=[
            pltpu.VMEM((K_INNER, CHUNK), jnp.int32),
            pltpu.VMEM((CHUNK, 1), jnp.float32),
            pltpu.VMEM_SHARED((N_PAD, 1), jnp.float32),
            pltpu.SemaphoreType.DMA,
        ],
    )
    def deg(ones_hbm, zeros_hbm, dst_hbm, out_hbm, dstb, onesb, acc, isem):
        c = lax.axis_index("c")
        s = lax.axis_index("s")
        wid = c * N_SUBCORES + s
        zr = N_PAD // N_SUBCORES
        pltpu.sync_copy(zeros_hbm.at[pl.ds(s * zr, zr)],
                        acc.at[pl.ds(s * zr, zr)])
        pltpu.sync_copy(ones_hbm, onesb)
        plsc.subcore_barrier()
        row0 = wid * R_PER_W

        def body(b, carry):
            r0 = row0 + b * K_INNER
            pltpu.async_copy(dst_hbm.at[pl.ds(r0, K_INNER)], dstb, isem).wait()
            for j in range(K_INNER):
                pltpu.sync_copy(onesb, acc.at[dstb.at[j]], add=True)
            return carry

        lax.fori_loop(0, N_BLK, body, 0)
        plsc.subcore_barrier()
        pltpu.sync_copy(acc.at[pl.ds(s * zr, zr)],
                        out_hbm.at[c, pl.ds(s * zr, zr)])

    return deg


_BR = 256
_GRID = N_PAD // _BR


def _row_spec(d):
    return pl.BlockSpec((_BR, d), lambda i: (i, 0))


def _full_spec(shape):
    return pl.BlockSpec(shape, lambda i: (0, 0))


def _tc1_body(d0, d1, xp, dinv, u1):
    dv = lax.rsqrt(1.0 + d0[...] + d1[...])
    dinv[...] = dv
    u1[...] = xp[...] * dv


def _tc2_body(p0, p1, u1, dv, w, b, u2):
    h = (p0[...] + p1[...] + u1[...]) * dv[...]
    z = jnp.dot(h, w[...], preferred_element_type=jnp.float32) + b[...][0:1, :]
    u2[...] = jnp.maximum(z, 0.0) * dv[...]


def _tc3_body(p0, p1, u2, dv, w3, b3, w2, u3):
    h = (p0[...] + p1[...] + u2[...]) * dv[...]
    z = jnp.dot(h, w3[...], preferred_element_type=jnp.float32) + b3[...][0:1, :]
    t = jnp.maximum(z, 0.0)
    u3[...] = jnp.dot(t, w2[...], preferred_element_type=jnp.float32) * dv[...]


_BR_OUT = 400
_GRID_OUT = N_NODES // _BR_OUT


def _tc4_body(p0, p1, u3, dv, b2, out):
    z = (p0[...] + p1[...] + u3[...]) * dv[...] + b2[...][0:1, :]
    col = lax.broadcasted_iota(jnp.int32, z.shape, 1)
    zm = jnp.where(col < 6, z, -1e30)
    m = jnp.max(zm, axis=1, keepdims=True)
    e = jnp.exp(zm - m)
    lse = jnp.log(jnp.sum(e, axis=1, keepdims=True)) + m
    out[...] = (z - lse)[:, :6]


def kernel(x, edge_index, W1, b1, W3, b3, W2, b2):
    f32 = jnp.float32
    src = edge_index[0].astype(jnp.int32)
    dst = edge_index[1].astype(jnp.int32)
    pad = jnp.full((E_PAD - E_EDGES,), N_NODES, jnp.int32)
    src2d = jnp.concatenate([src, pad]).reshape(E_ROWS, CHUNK)
    dst2d = jnp.concatenate([dst, pad]).reshape(E_ROWS, CHUNK)

    x_pad = jnp.zeros((N_PAD, 8), f32).at[:N_NODES, :6].set(x)
    w1p = jnp.zeros((8, 16), f32).at[:6, :].set(W1)
    b1p = jnp.zeros((8, 16), f32).at[0, :].set(b1)
    b3p = jnp.zeros((8, 16), f32).at[0, :].set(b3)
    w2p = jnp.zeros((16, 8), f32).at[:, :6].set(W2)
    b2p = jnp.zeros((8, 8), f32).at[0, :6].set(b2)

    zeros1 = jnp.zeros((N_PAD, 1), f32)
    zeros8 = jnp.zeros((N_PAD, 8), f32)
    zeros16 = jnp.zeros((N_PAD, 16), f32)
    ones = jnp.ones((CHUNK, 1), f32)

    degp = _make_deg()(ones, zeros1, dst2d)

    dinv, u1 = pl.pallas_call(
        _tc1_body,
        grid=(_GRID,),
        in_specs=[_row_spec(1), _row_spec(1), _row_spec(8)],
        out_specs=[_row_spec(1), _row_spec(8)],
        out_shape=[jax.ShapeDtypeStruct((N_PAD, 1), f32),
                   jax.ShapeDtypeStruct((N_PAD, 8), f32)],
    )(degp[0], degp[1], x_pad)

    p1 = _make_prop(8)(u1, zeros8, src2d, dst2d)

    u2 = pl.pallas_call(
        _tc2_body,
        grid=(_GRID,),
        in_specs=[_row_spec(8), _row_spec(8), _row_spec(8), _row_spec(1),
                  _full_spec((8, 16)), _full_spec((8, 16))],
        out_specs=_row_spec(16),
        out_shape=jax.ShapeDtypeStruct((N_PAD, 16), f32),
    )(p1[0], p1[1], u1, dinv, w1p, b1p)

    p2 = _make_prop(16)(u2, zeros16, src2d, dst2d)

    u3 = pl.pallas_call(
        _tc3_body,
        grid=(_GRID,),
        in_specs=[_row_spec(16), _row_spec(16), _row_spec(16), _row_spec(1),
                  _full_spec((16, 16)), _full_spec((8, 16)),
                  _full_spec((16, 8))],
        out_specs=_row_spec(8),
        out_shape=jax.ShapeDtypeStruct((N_PAD, 8), f32),
    )(p2[0], p2[1], u2, dinv, W3, b3p, w2p)

    p3 = _make_prop(8)(u3, zeros8, src2d, dst2d)

    out = pl.pallas_call(
        _tc4_body,
        grid=(_GRID_OUT,),
        in_specs=[pl.BlockSpec((_BR_OUT, 8), lambda i: (i, 0)),
                  pl.BlockSpec((_BR_OUT, 8), lambda i: (i, 0)),
                  pl.BlockSpec((_BR_OUT, 8), lambda i: (i, 0)),
                  pl.BlockSpec((_BR_OUT, 1), lambda i: (i, 0)),
                  _full_spec((8, 8))],
        out_specs=pl.BlockSpec((_BR_OUT, 6), lambda i: (i, 0)),
        out_shape=jax.ShapeDtypeStruct((N_NODES, 6), f32),
    )(p3[0], p3[1], u3, dinv, b2p)

    return out
